# baseline (speedup 1.0000x reference)
import functools
import math

import jax
import jax.numpy as jnp
from jax import lax
from jax.experimental import pallas as pl
from jax.experimental.pallas import tpu as pltpu
from jax.experimental.pallas import tpu_sc as plsc

B, T, D = 2, 4096, 1024
H = 4096
E = 8
N = B * T
CAP = max(4, math.ceil(1.0 * N / E))
Z_COEF = 1e-3
AUX_COEF = 1e-2

NB = 8
TB = N // NB
HK = 8
HB = H // HK

NW = 32
TW = N // NW
RCH = 64
NCH = TW // RCH


def _router_body(x_ref, gw_ref, eidx_ref, rank_ref, p_ref, meta_ref, aux_ref,
                 carry, psum, zsum):
    b = pl.program_id(0)

    @pl.when(b == 0)
    def _():
        carry[...] = jnp.zeros_like(carry)
        psum[...] = jnp.zeros_like(psum)
        zsum[...] = jnp.zeros_like(zsum)

    xb = x_ref[...]
    gw = gw_ref[...]
    logits = lax.dot_general(gw, xb, (((1,), (1,)), ((), ())),
                             precision="default",
                             preferred_element_type=jnp.float32)
    m = jnp.max(logits, axis=0, keepdims=True)
    ex = jnp.exp(logits - m)
    se = jnp.sum(ex, axis=0, keepdims=True)
    probs = ex / se
    lse = m + jnp.log(se)
    zsum[...] += jnp.sum(lse * lse, axis=(0, 1), keepdims=True)
    psum[...] += jnp.sum(probs, axis=1, keepdims=True)

    ids = lax.broadcasted_iota(jnp.int32, (E, TB), 0)
    eq = logits == m
    eidx = jnp.min(jnp.where(eq, ids, E), axis=0, keepdims=True)
    prow = jnp.max(probs, axis=0, keepdims=True)

    onehot = (ids == eidx).astype(jnp.float32)
    ri = lax.broadcasted_iota(jnp.int32, (TB, TB), 0)
    ci = lax.broadcasted_iota(jnp.int32, (TB, TB), 1)
    tri = (ri < ci).astype(jnp.bfloat16)
    rank_mat = lax.dot_general(onehot.astype(jnp.bfloat16), tri,
                               (((1,), (0,)), ((), ())),
                               preferred_element_type=jnp.float32)
    rank_tot = rank_mat + carry[...]
    rank = jnp.sum(rank_tot * onehot, axis=0, keepdims=True)
    carry[...] += jnp.sum(onehot, axis=1, keepdims=True)

    eidx_ref[...] = eidx
    rank_ref[...] = rank.astype(jnp.int32)
    p_ref[...] = prow

    @pl.when(b == NB - 1)
    def _():
        counts = carry[...]
        e1 = lax.broadcasted_iota(jnp.int32, (E, E), 0)
        e2 = lax.broadcasted_iota(jnp.int32, (E, E), 1)
        m8 = (e2 < e1).astype(jnp.float32)
        start = lax.dot_general(m8, counts, (((1,), (0,)), ((), ())),
                                precision=lax.Precision.HIGHEST,
                                preferred_element_type=jnp.float32)
        ids8 = lax.broadcasted_iota(jnp.int32, (E, 1), 0)
        cmin = jnp.min(counts, axis=0, keepdims=True)
        emin = jnp.min(jnp.where(counts == cmin, ids8, E), axis=0,
                       keepdims=True)
        zslot = emin * CAP + (CAP - 1)
        pad = lax.broadcasted_iota(jnp.int32, (16, 1), 0)
        padv = jnp.where(pad == 0, zslot, 0)
        meta_ref[...] = jnp.concatenate(
            [start.astype(jnp.int32), counts.astype(jnp.int32), padv], axis=0)
        fp = jnp.sum((counts / N) * (psum[...] / N), axis=(0, 1), keepdims=True)
        aux_ref[...] = AUX_COEF * E * fp + Z_COEF * (zsum[...] / N)


def _router(x_flat, gate_w):
    return pl.pallas_call(
        _router_body,
        grid=(NB,),
        in_specs=[
            pl.BlockSpec((TB, D), lambda b: (b, 0)),
            pl.BlockSpec((E, D), lambda b: (0, 0)),
        ],
        out_specs=[
            pl.BlockSpec((None, 1, TB), lambda b: (b, 0, 0)),
            pl.BlockSpec((None, 1, TB), lambda b: (b, 0, 0)),
            pl.BlockSpec((None, 1, TB), lambda b: (b, 0, 0)),
            pl.BlockSpec((32, 1), lambda b: (0, 0)),
            pl.BlockSpec((1, 1), lambda b: (0, 0)),
        ],
        out_shape=[
            jax.ShapeDtypeStruct((NB, 1, TB), jnp.int32),
            jax.ShapeDtypeStruct((NB, 1, TB), jnp.int32),
            jax.ShapeDtypeStruct((NB, 1, TB), jnp.float32),
            jax.ShapeDtypeStruct((32, 1), jnp.int32),
            jax.ShapeDtypeStruct((1, 1), jnp.float32),
        ],
        scratch_shapes=[
            pltpu.VMEM((E, 1), jnp.float32),
            pltpu.VMEM((E, 1), jnp.float32),
            pltpu.VMEM((1, 1), jnp.float32),
        ],
    )(x_flat, gate_w)


def _sc_wid():
    return lax.axis_index("s") * 2 + lax.axis_index("c")


def _scatter_body(eidx_hbm, rank_hbm, p_hbm, meta_hbm,
                  sidx_hbm, q_hbm, g_hbm,
                  ev, rv, pv, metav, spv, tokv, gv, sem):
    wid = _sc_wid()
    base = wid * TW
    d0 = pltpu.async_copy(eidx_hbm.at[pl.ds(base, TW)], ev, sem)
    d1 = pltpu.async_copy(rank_hbm.at[pl.ds(base, TW)], rv, sem)
    d2 = pltpu.async_copy(p_hbm.at[pl.ds(base, TW)], pv, sem)
    d3 = pltpu.async_copy(meta_hbm, metav, sem)
    d0.wait(); d1.wait(); d2.wait(); d3.wait()

    iota = lax.iota(jnp.int32, 16)
    z16 = jnp.full((16,), 16, jnp.int32)
    zs16 = plsc.load_gather(metav, [z16])
    for i in range(TW // 16):
        sl = pl.ds(i * 16, 16)
        e16 = ev[sl]
        r16 = rv[sl]
        st16 = plsc.load_gather(metav, [e16])
        sp16 = st16 + r16
        c16 = e16 * CAP + r16
        gv[sl] = jnp.where(r16 < CAP, c16, zs16)
        spv[i // 8, pl.ds((i % 8) * 16, 16)] = sp16
        tokv[sl] = base + i * 16 + iota

    w0 = pltpu.async_copy(gv, g_hbm.at[pl.ds(base, TW)], sem)
    ws = []
    for j in range(TW // 128):
        ws.append(pltpu.async_copy(tokv.at[pl.ds(j * 128, 128)],
                                   sidx_hbm.at[spv.at[j]], sem))
        ws.append(pltpu.async_copy(pv.at[pl.ds(j * 128, 128)],
                                   q_hbm.at[spv.at[j]], sem))
    w0.wait()
    for w in ws:
        w.wait()


def _sc_scatter(eidx, rank, p, meta):
    f = pl.kernel(
        _scatter_body,
        out_type=[
            jax.ShapeDtypeStruct((N,), jnp.int32),
            jax.ShapeDtypeStruct((N,), jnp.float32),
            jax.ShapeDtypeStruct((N,), jnp.int32),
        ],
        mesh=plsc.VectorSubcoreMesh(core_axis_name="c", subcore_axis_name="s", num_cores=2, num_subcores=16),
        compiler_params=pltpu.CompilerParams(needs_layout_passes=False),
        scratch_types=[
            pltpu.VMEM((TW,), jnp.int32),
            pltpu.VMEM((TW,), jnp.int32),
            pltpu.VMEM((TW,), jnp.float32),
            pltpu.VMEM((32,), jnp.int32),
            pltpu.VMEM((TW // 128, 128), jnp.int32),
            pltpu.VMEM((TW,), jnp.int32),
            pltpu.VMEM((TW,), jnp.int32),
            pltpu.SemaphoreType.DMA,
        ],
    )
    return f(eidx, rank, p, meta)


DW = D // 2


def _dispatch_body(x_hbm, sidx_hbm, q_hbm, meta_hbm,
                   xd_hbm, scale_hbm,
                   metav, jv, tv, qv, scv, rb0, rb1, sem, gs0, gs1):
    wid = _sc_wid()
    slot_base = wid * TW
    e = slot_base // CAP
    rbase = slot_base - e * CAP
    pltpu.sync_copy(meta_hbm, metav)

    iota = lax.iota(jnp.int32, 16)
    e16 = jnp.zeros((16,), jnp.int32) + e
    st16 = plsc.load_gather(metav, [e16])
    cn16 = plsc.load_gather(metav, [e16 + 8])
    for i in range(TW // 16):
        sl = pl.ds(i * 16, 16)
        r16 = rbase + i * 16 + iota
        jm = st16 + jnp.minimum(r16, jnp.maximum(cn16 - 1, 0))
        jv[sl] = jnp.minimum(jm, N - 1)

    for j in range(TW // 128):
        sl = pl.ds(j * 128, 128)
        pltpu.sync_copy(sidx_hbm.at[jv.at[sl]], tv.at[sl])
    for j in range(TW // 128):
        sl = pl.ds(j * 128, 128)
        pltpu.sync_copy(q_hbm.at[tv.at[sl]], qv.at[sl])

    for i in range(TW // 16):
        sl = pl.ds(i * 16, 16)
        r16 = rbase + i * 16 + iota
        scv[sl] = jnp.where(r16 < cn16, qv[sl], 0.0)
    wsc = pltpu.async_copy(scv, scale_hbm.at[pl.ds(slot_base, TW)], sem)

    bufs = (rb0, rb1)
    sems = (gs0, gs1)
    descs = [None, None]
    descs[0] = pltpu.async_copy(x_hbm.at[tv.at[pl.ds(0, RCH)]], bufs[0], gs0)
    for ch in range(NCH):
        nxt = ch + 1
        if nxt < NCH:
            descs[nxt % 2] = pltpu.async_copy(
                x_hbm.at[tv.at[pl.ds(nxt * RCH, RCH)]], bufs[nxt % 2],
                sems[nxt % 2])
        descs[ch % 2].wait()
        pltpu.sync_copy(bufs[ch % 2],
                        xd_hbm.at[pl.ds(slot_base + ch * RCH, RCH)])
    wsc.wait()


def _sc_dispatch(x_bf, sidx, q, meta):
    f = pl.kernel(
        _dispatch_body,
        out_type=[
            jax.ShapeDtypeStruct((N, DW), jnp.int32),
            jax.ShapeDtypeStruct((N,), jnp.float32),
        ],
        mesh=plsc.VectorSubcoreMesh(core_axis_name="c", subcore_axis_name="s", num_cores=2, num_subcores=16),
        compiler_params=pltpu.CompilerParams(needs_layout_passes=False),
        scratch_types=[
            pltpu.VMEM((32,), jnp.int32),
            pltpu.VMEM((TW,), jnp.int32),
            pltpu.VMEM((TW,), jnp.int32),
            pltpu.VMEM((TW,), jnp.float32),
            pltpu.VMEM((TW,), jnp.float32),
            pltpu.VMEM((RCH, DW), jnp.int32),
            pltpu.VMEM((RCH, DW), jnp.int32),
            pltpu.SemaphoreType.DMA,
            pltpu.SemaphoreType.DMA,
            pltpu.SemaphoreType.DMA,
        ],
    )
    return f(x_bf, sidx, q, meta)


def _gelu(v):
    return 0.5 * v * (1.0 + lax.erf(v * (1.0 / math.sqrt(2.0))))


def _ffn_body(xd_ref, w1_ref, b1_ref, w2_ref, b2_ref, sc_ref, out_ref,
              acc):
    h = pl.program_id(1)

    @pl.when(h == 0)
    def _():
        acc[...] = jnp.broadcast_to(b2_ref[0], (CAP, D))

    hm = lax.dot_general(xd_ref[...], w1_ref[...], (((1,), (1,)), ((), ())),
                         preferred_element_type=jnp.float32)
    hm = hm + b1_ref[0]
    hg = _gelu(hm).astype(jnp.bfloat16)
    acc[...] += lax.dot_general(hg, w2_ref[...], (((1,), (1,)), ((), ())),
                                preferred_element_type=jnp.float32)

    @pl.when(h == HK - 1)
    def _():
        out_ref[...] = acc[...] * sc_ref[...]


def _ffn(xd, w1, b1, w2, b2, scale):
    return pl.pallas_call(
        _ffn_body,
        grid=(E, HK),
        in_specs=[
            pl.BlockSpec((None, CAP, D), lambda e, h: (e, 0, 0)),
            pl.BlockSpec((None, HB, D), lambda e, h: (e, h, 0)),
            pl.BlockSpec((None, 1, HB), lambda e, h: (e, 0, h)),
            pl.BlockSpec((None, D, HB), lambda e, h: (e, 0, h)),
            pl.BlockSpec((None, 1, D), lambda e, h: (e, 0, 0)),
            pl.BlockSpec((None, CAP, 1), lambda e, h: (e, 0, 0)),
        ],
        out_specs=pl.BlockSpec((None, CAP, D), lambda e, h: (e, 0, 0)),
        out_shape=jax.ShapeDtypeStruct((E, CAP, D), jnp.float32),
        scratch_shapes=[
            pltpu.VMEM((CAP, D), jnp.float32),
        ],
    )(xd, w1, b1, w2, b2, scale)


RC5 = 32
NC5 = TW // RC5


def _combine_body(od_hbm, g_hbm, out_hbm, gv, rb0, rb1, gs0, gs1):
    wid = _sc_wid()
    base = wid * TW
    pltpu.sync_copy(g_hbm.at[pl.ds(base, TW)], gv)
    bufs = (rb0, rb1)
    sems = (gs0, gs1)
    descs = [None, None]
    descs[0] = pltpu.async_copy(od_hbm.at[gv.at[pl.ds(0, RC5)]], bufs[0], gs0)
    for ch in range(NC5):
        nxt = ch + 1
        if nxt < NC5:
            descs[nxt % 2] = pltpu.async_copy(
                od_hbm.at[gv.at[pl.ds(nxt * RC5, RC5)]], bufs[nxt % 2],
                sems[nxt % 2])
        descs[ch % 2].wait()
        pltpu.sync_copy(bufs[ch % 2], out_hbm.at[pl.ds(base + ch * RC5, RC5)])


def _sc_combine(od, g):
    f = pl.kernel(
        _combine_body,
        out_type=jax.ShapeDtypeStruct((N, D), jnp.float32),
        mesh=plsc.VectorSubcoreMesh(core_axis_name="c", subcore_axis_name="s", num_cores=2, num_subcores=16),
        compiler_params=pltpu.CompilerParams(needs_layout_passes=False),
        scratch_types=[
            pltpu.VMEM((TW,), jnp.int32),
            pltpu.VMEM((RC5, D), jnp.float32),
            pltpu.VMEM((RC5, D), jnp.float32),
            pltpu.SemaphoreType.DMA,
            pltpu.SemaphoreType.DMA,
        ],
    )
    return f(od, g)


@jax.jit
def kernel(x, gate_w, W1, b1, W2, b2):
    x_flat = x.reshape(N, D)
    eidx, rank, p, meta, aux = _router(x_flat, gate_w)
    sidx, q, g = _sc_scatter(eidx.reshape(N), rank.reshape(N),
                             p.reshape(N), meta.reshape(32))
    x_pack = lax.bitcast_convert_type(
        x_flat.astype(jnp.bfloat16).reshape(N, D // 2, 2), jnp.int32)
    x_disp, scale = _sc_dispatch(x_pack, sidx, q, meta.reshape(32))
    x_disp = lax.bitcast_convert_type(x_disp, jnp.bfloat16).reshape(N, D)
    od = _ffn(x_disp.reshape(E, CAP, D),
              W1.astype(jnp.bfloat16), b1.reshape(E, 1, H),
              W2.astype(jnp.bfloat16), b2.reshape(E, 1, D),
              scale.reshape(E, CAP, 1))
    out = _sc_combine(od.reshape(N, D), g)
    return out.reshape(B, T, D), aux.reshape(())[()]


# speedup vs baseline: 1.6779x; 1.6779x over previous
import functools
import math

import jax
import jax.numpy as jnp
from jax import lax
from jax.experimental import pallas as pl
from jax.experimental.pallas import tpu as pltpu
from jax.experimental.pallas import tpu_sc as plsc

B, T, D = 2, 4096, 1024
H = 4096
E = 8
N = B * T
CAP = max(4, math.ceil(1.0 * N / E))
Z_COEF = 1e-3
AUX_COEF = 1e-2

NB = 8
TB = N // NB
HK = 8
HB = H // HK

NW = 32
TW = N // NW
RCH = 32
NCH = TW // RCH


def _router_body(x_ref, gw_ref, eidx_ref, rank_ref, p_ref, meta_ref, aux_ref,
                 carry, psum, zsum):
    b = pl.program_id(0)

    @pl.when(b == 0)
    def _():
        carry[...] = jnp.zeros_like(carry)
        psum[...] = jnp.zeros_like(psum)
        zsum[...] = jnp.zeros_like(zsum)

    xb = x_ref[...]
    gw = gw_ref[...]
    logits = lax.dot_general(gw, xb, (((1,), (1,)), ((), ())),
                             precision="default",
                             preferred_element_type=jnp.float32)
    m = jnp.max(logits, axis=0, keepdims=True)
    ex = jnp.exp(logits - m)
    se = jnp.sum(ex, axis=0, keepdims=True)
    probs = ex / se
    lse = m + jnp.log(se)
    zsum[...] += jnp.sum(lse * lse, axis=(0, 1), keepdims=True)
    psum[...] += jnp.sum(probs, axis=1, keepdims=True)

    ids = lax.broadcasted_iota(jnp.int32, (E, TB), 0)
    eq = logits == m
    eidx = jnp.min(jnp.where(eq, ids, E), axis=0, keepdims=True)
    prow = jnp.max(probs, axis=0, keepdims=True)

    onehot = (ids == eidx).astype(jnp.float32)
    ri = lax.broadcasted_iota(jnp.int32, (TB, TB), 0)
    ci = lax.broadcasted_iota(jnp.int32, (TB, TB), 1)
    tri = (ri < ci).astype(jnp.bfloat16)
    rank_mat = lax.dot_general(onehot.astype(jnp.bfloat16), tri,
                               (((1,), (0,)), ((), ())),
                               preferred_element_type=jnp.float32)
    rank_tot = rank_mat + carry[...]
    rank = jnp.sum(rank_tot * onehot, axis=0, keepdims=True)
    carry[...] += jnp.sum(onehot, axis=1, keepdims=True)

    eidx_ref[...] = eidx
    rank_ref[...] = rank.astype(jnp.int32)
    p_ref[...] = prow

    @pl.when(b == NB - 1)
    def _():
        counts = carry[...]
        e1 = lax.broadcasted_iota(jnp.int32, (E, E), 0)
        e2 = lax.broadcasted_iota(jnp.int32, (E, E), 1)
        m8 = (e2 < e1).astype(jnp.float32)
        start = lax.dot_general(m8, counts, (((1,), (0,)), ((), ())),
                                precision=lax.Precision.HIGHEST,
                                preferred_element_type=jnp.float32)
        ids8 = lax.broadcasted_iota(jnp.int32, (E, 1), 0)
        cmin = jnp.min(counts, axis=0, keepdims=True)
        emin = jnp.min(jnp.where(counts == cmin, ids8, E), axis=0,
                       keepdims=True)
        zslot = emin * CAP + (CAP - 1)
        pad = lax.broadcasted_iota(jnp.int32, (16, 1), 0)
        padv = jnp.where(pad == 0, zslot, 0)
        meta_ref[...] = jnp.concatenate(
            [start.astype(jnp.int32), counts.astype(jnp.int32), padv], axis=0)
        fp = jnp.sum((counts / N) * (psum[...] / N), axis=(0, 1), keepdims=True)
        aux_ref[...] = AUX_COEF * E * fp + Z_COEF * (zsum[...] / N)


def _router(x_flat, gate_w):
    return pl.pallas_call(
        _router_body,
        grid=(NB,),
        in_specs=[
            pl.BlockSpec((TB, D), lambda b: (b, 0)),
            pl.BlockSpec((E, D), lambda b: (0, 0)),
        ],
        out_specs=[
            pl.BlockSpec((None, 1, TB), lambda b: (b, 0, 0)),
            pl.BlockSpec((None, 1, TB), lambda b: (b, 0, 0)),
            pl.BlockSpec((None, 1, TB), lambda b: (b, 0, 0)),
            pl.BlockSpec((32, 1), lambda b: (0, 0)),
            pl.BlockSpec((1, 1), lambda b: (0, 0)),
        ],
        out_shape=[
            jax.ShapeDtypeStruct((NB, 1, TB), jnp.int32),
            jax.ShapeDtypeStruct((NB, 1, TB), jnp.int32),
            jax.ShapeDtypeStruct((NB, 1, TB), jnp.float32),
            jax.ShapeDtypeStruct((32, 1), jnp.int32),
            jax.ShapeDtypeStruct((1, 1), jnp.float32),
        ],
        scratch_shapes=[
            pltpu.VMEM((E, 1), jnp.float32),
            pltpu.VMEM((E, 1), jnp.float32),
            pltpu.VMEM((1, 1), jnp.float32),
        ],
    )(x_flat, gate_w)


def _sc_wid():
    return lax.axis_index("s") * 2 + lax.axis_index("c")


def _scatter_body(eidx_hbm, rank_hbm, p_hbm, meta_hbm,
                  sidx_hbm, q_hbm, g_hbm,
                  ev, rv, pv, metav, spv, tokv, gv, sem):
    wid = _sc_wid()
    base = wid * TW
    d0 = pltpu.async_copy(eidx_hbm.at[pl.ds(base, TW)], ev, sem)
    d1 = pltpu.async_copy(rank_hbm.at[pl.ds(base, TW)], rv, sem)
    d2 = pltpu.async_copy(p_hbm.at[pl.ds(base, TW)], pv, sem)
    d3 = pltpu.async_copy(meta_hbm, metav, sem)
    d0.wait(); d1.wait(); d2.wait(); d3.wait()

    iota = lax.iota(jnp.int32, 16)
    z16 = jnp.full((16,), 16, jnp.int32)
    zs16 = plsc.load_gather(metav, [z16])
    for i in range(TW // 16):
        sl = pl.ds(i * 16, 16)
        e16 = ev[sl]
        r16 = rv[sl]
        st16 = plsc.load_gather(metav, [e16])
        sp16 = st16 + r16
        c16 = e16 * CAP + r16
        gv[sl] = jnp.where(r16 < CAP, c16, zs16)
        spv[i // 8, pl.ds((i % 8) * 16, 16)] = sp16
        tokv[sl] = base + i * 16 + iota

    w0 = pltpu.async_copy(gv, g_hbm.at[pl.ds(base, TW)], sem)
    ws = []
    for j in range(TW // 128):
        ws.append(pltpu.async_copy(tokv.at[pl.ds(j * 128, 128)],
                                   sidx_hbm.at[spv.at[j]], sem))
        ws.append(pltpu.async_copy(pv.at[pl.ds(j * 128, 128)],
                                   q_hbm.at[spv.at[j]], sem))
    w0.wait()
    for w in ws:
        w.wait()


def _sc_scatter(eidx, rank, p, meta):
    f = pl.kernel(
        _scatter_body,
        out_type=[
            jax.ShapeDtypeStruct((N,), jnp.int32),
            jax.ShapeDtypeStruct((N,), jnp.float32),
            jax.ShapeDtypeStruct((N,), jnp.int32),
        ],
        mesh=plsc.VectorSubcoreMesh(core_axis_name="c", subcore_axis_name="s", num_cores=2, num_subcores=16),
        compiler_params=pltpu.CompilerParams(needs_layout_passes=False),
        scratch_types=[
            pltpu.VMEM((TW,), jnp.int32),
            pltpu.VMEM((TW,), jnp.int32),
            pltpu.VMEM((TW,), jnp.float32),
            pltpu.VMEM((32,), jnp.int32),
            pltpu.VMEM((TW // 128, 128), jnp.int32),
            pltpu.VMEM((TW,), jnp.int32),
            pltpu.VMEM((TW,), jnp.int32),
            pltpu.SemaphoreType.DMA,
        ],
    )
    return f(eidx, rank, p, meta)


def _dispatch_body(x_hbm, sidx_hbm, q_hbm, meta_hbm,
                   xd_hbm, scale_hbm,
                   metav, jv, tv, qv, scv, rb0, rb1, sem, gs0, gs1):
    wid = _sc_wid()
    slot_base = wid * TW
    e = slot_base // CAP
    rbase = slot_base - e * CAP
    pltpu.sync_copy(meta_hbm, metav)

    iota = lax.iota(jnp.int32, 16)
    e16 = jnp.zeros((16,), jnp.int32) + e
    st16 = plsc.load_gather(metav, [e16])
    cn16 = plsc.load_gather(metav, [e16 + 8])
    for i in range(TW // 16):
        sl = pl.ds(i * 16, 16)
        r16 = rbase + i * 16 + iota
        jm = st16 + jnp.minimum(r16, jnp.maximum(cn16 - 1, 0))
        jv[sl] = jnp.minimum(jm, N - 1)

    for j in range(TW // 128):
        sl = pl.ds(j * 128, 128)
        pltpu.sync_copy(sidx_hbm.at[jv.at[sl]], tv.at[sl])
    for j in range(TW // 128):
        sl = pl.ds(j * 128, 128)
        pltpu.sync_copy(q_hbm.at[tv.at[sl]], qv.at[sl])

    for i in range(TW // 16):
        sl = pl.ds(i * 16, 16)
        r16 = rbase + i * 16 + iota
        scv[sl] = jnp.where(r16 < cn16, qv[sl], 0.0)
    wsc = pltpu.async_copy(scv, scale_hbm.at[pl.ds(slot_base, TW)], sem)

    bufs = (rb0, rb1)
    sems = (gs0, gs1)
    descs = [None, None]
    descs[0] = pltpu.async_copy(x_hbm.at[tv.at[pl.ds(0, RCH)]], bufs[0], gs0)
    for ch in range(NCH):
        nxt = ch + 1
        if nxt < NCH:
            descs[nxt % 2] = pltpu.async_copy(
                x_hbm.at[tv.at[pl.ds(nxt * RCH, RCH)]], bufs[nxt % 2],
                sems[nxt % 2])
        descs[ch % 2].wait()
        pltpu.sync_copy(bufs[ch % 2],
                        xd_hbm.at[pl.ds(slot_base + ch * RCH, RCH)])
    wsc.wait()


def _sc_dispatch(x_bf, sidx, q, meta):
    f = pl.kernel(
        _dispatch_body,
        out_type=[
            jax.ShapeDtypeStruct((N, D), jnp.float32),
            jax.ShapeDtypeStruct((N,), jnp.float32),
        ],
        mesh=plsc.VectorSubcoreMesh(core_axis_name="c", subcore_axis_name="s", num_cores=2, num_subcores=16),
        compiler_params=pltpu.CompilerParams(needs_layout_passes=False),
        scratch_types=[
            pltpu.VMEM((32,), jnp.int32),
            pltpu.VMEM((TW,), jnp.int32),
            pltpu.VMEM((TW,), jnp.int32),
            pltpu.VMEM((TW,), jnp.float32),
            pltpu.VMEM((TW,), jnp.float32),
            pltpu.VMEM((RCH, D), jnp.float32),
            pltpu.VMEM((RCH, D), jnp.float32),
            pltpu.SemaphoreType.DMA,
            pltpu.SemaphoreType.DMA,
            pltpu.SemaphoreType.DMA,
        ],
    )
    return f(x_bf, sidx, q, meta)


def _gelu(v):
    return 0.5 * v * (1.0 + lax.erf(v * (1.0 / math.sqrt(2.0))))


def _ffn_body(xd_ref, w1_ref, b1_ref, w2_ref, b2_ref, sc_ref, out_ref,
              xbf, acc):
    h = pl.program_id(1)

    @pl.when(h == 0)
    def _():
        xbf[...] = xd_ref[...].astype(jnp.bfloat16)
        acc[...] = jnp.broadcast_to(b2_ref[0], (CAP, D))

    hm = lax.dot_general(xbf[...], w1_ref[...], (((1,), (1,)), ((), ())),
                         preferred_element_type=jnp.float32)
    hm = hm + b1_ref[0]
    hg = _gelu(hm).astype(jnp.bfloat16)
    acc[...] += lax.dot_general(hg, w2_ref[...], (((1,), (1,)), ((), ())),
                                preferred_element_type=jnp.float32)

    @pl.when(h == HK - 1)
    def _():
        out_ref[...] = acc[...] * sc_ref[...]


def _ffn(xd, w1, b1, w2, b2, scale):
    return pl.pallas_call(
        _ffn_body,
        grid=(E, HK),
        in_specs=[
            pl.BlockSpec((None, CAP, D), lambda e, h: (e, 0, 0)),
            pl.BlockSpec((None, HB, D), lambda e, h: (e, h, 0)),
            pl.BlockSpec((None, 1, HB), lambda e, h: (e, 0, h)),
            pl.BlockSpec((None, D, HB), lambda e, h: (e, 0, h)),
            pl.BlockSpec((None, 1, D), lambda e, h: (e, 0, 0)),
            pl.BlockSpec((None, CAP, 1), lambda e, h: (e, 0, 0)),
        ],
        out_specs=pl.BlockSpec((None, CAP, D), lambda e, h: (e, 0, 0)),
        out_shape=jax.ShapeDtypeStruct((E, CAP, D), jnp.float32),
        scratch_shapes=[
            pltpu.VMEM((CAP, D), jnp.bfloat16),
            pltpu.VMEM((CAP, D), jnp.float32),
        ],
    )(xd, w1, b1, w2, b2, scale)


RC5 = 32
NC5 = TW // RC5


def _combine_body(od_hbm, g_hbm, out_hbm, gv, rb0, rb1, gs0, gs1):
    wid = _sc_wid()
    base = wid * TW
    pltpu.sync_copy(g_hbm.at[pl.ds(base, TW)], gv)
    bufs = (rb0, rb1)
    sems = (gs0, gs1)
    descs = [None, None]
    descs[0] = pltpu.async_copy(od_hbm.at[gv.at[pl.ds(0, RC5)]], bufs[0], gs0)
    for ch in range(NC5):
        nxt = ch + 1
        if nxt < NC5:
            descs[nxt % 2] = pltpu.async_copy(
                od_hbm.at[gv.at[pl.ds(nxt * RC5, RC5)]], bufs[nxt % 2],
                sems[nxt % 2])
        descs[ch % 2].wait()
        pltpu.sync_copy(bufs[ch % 2], out_hbm.at[pl.ds(base + ch * RC5, RC5)])


def _sc_combine(od, g):
    f = pl.kernel(
        _combine_body,
        out_type=jax.ShapeDtypeStruct((N, D), jnp.float32),
        mesh=plsc.VectorSubcoreMesh(core_axis_name="c", subcore_axis_name="s", num_cores=2, num_subcores=16),
        compiler_params=pltpu.CompilerParams(needs_layout_passes=False),
        scratch_types=[
            pltpu.VMEM((TW,), jnp.int32),
            pltpu.VMEM((RC5, D), jnp.float32),
            pltpu.VMEM((RC5, D), jnp.float32),
            pltpu.SemaphoreType.DMA,
            pltpu.SemaphoreType.DMA,
        ],
    )
    return f(od, g)


@jax.jit
def kernel(x, gate_w, W1, b1, W2, b2):
    x_flat = x.reshape(N, D)
    eidx, rank, p, meta, aux = _router(x_flat, gate_w)
    sidx, q, g = _sc_scatter(eidx.reshape(N), rank.reshape(N),
                             p.reshape(N), meta.reshape(32))
    x_disp, scale = _sc_dispatch(x_flat, sidx, q, meta.reshape(32))
    od = _ffn(x_disp.reshape(E, CAP, D),
              W1.astype(jnp.bfloat16), b1.reshape(E, 1, H),
              W2.astype(jnp.bfloat16), b2.reshape(E, 1, D),
              scale.reshape(E, CAP, 1))
    out = _sc_combine(od.reshape(N, D), g)
    return out.reshape(B, T, D), aux.reshape(())[()]


# speedup vs baseline: 1.8279x; 1.0894x over previous
import functools
import math

import jax
import jax.numpy as jnp
from jax import lax
from jax.experimental import pallas as pl
from jax.experimental.pallas import tpu as pltpu
from jax.experimental.pallas import tpu_sc as plsc

B, T, D = 2, 4096, 1024
H = 4096
E = 8
N = B * T
CAP = max(4, math.ceil(1.0 * N / E))
Z_COEF = 1e-3
AUX_COEF = 1e-2

NB = 8
TB = N // NB
HK = 1
HB = H // HK

NW = 32
TW = N // NW
RCH = 32
NCH = TW // RCH


def _router_body(x_ref, gw_ref, eidx_ref, rank_ref, p_ref, meta_ref, aux_ref,
                 carry, psum, zsum):
    b = pl.program_id(0)

    @pl.when(b == 0)
    def _():
        carry[...] = jnp.zeros_like(carry)
        psum[...] = jnp.zeros_like(psum)
        zsum[...] = jnp.zeros_like(zsum)

    xb = x_ref[...]
    gw = gw_ref[...]
    logits = lax.dot_general(gw, xb, (((1,), (1,)), ((), ())),
                             precision="default",
                             preferred_element_type=jnp.float32)
    m = jnp.max(logits, axis=0, keepdims=True)
    ex = jnp.exp(logits - m)
    se = jnp.sum(ex, axis=0, keepdims=True)
    probs = ex / se
    lse = m + jnp.log(se)
    zsum[...] += jnp.sum(lse * lse, axis=(0, 1), keepdims=True)
    psum[...] += jnp.sum(probs, axis=1, keepdims=True)

    ids = lax.broadcasted_iota(jnp.int32, (E, TB), 0)
    eq = logits == m
    eidx = jnp.min(jnp.where(eq, ids, E), axis=0, keepdims=True)
    prow = jnp.max(probs, axis=0, keepdims=True)

    onehot = (ids == eidx).astype(jnp.float32)
    ri = lax.broadcasted_iota(jnp.int32, (TB, TB), 0)
    ci = lax.broadcasted_iota(jnp.int32, (TB, TB), 1)
    tri = (ri < ci).astype(jnp.bfloat16)
    rank_mat = lax.dot_general(onehot.astype(jnp.bfloat16), tri,
                               (((1,), (0,)), ((), ())),
                               preferred_element_type=jnp.float32)
    rank_tot = rank_mat + carry[...]
    rank = jnp.sum(rank_tot * onehot, axis=0, keepdims=True)
    carry[...] += jnp.sum(onehot, axis=1, keepdims=True)

    eidx_ref[...] = eidx
    rank_ref[...] = rank.astype(jnp.int32)
    p_ref[...] = prow

    @pl.when(b == NB - 1)
    def _():
        counts = carry[...]
        e1 = lax.broadcasted_iota(jnp.int32, (E, E), 0)
        e2 = lax.broadcasted_iota(jnp.int32, (E, E), 1)
        m8 = (e2 < e1).astype(jnp.float32)
        start = lax.dot_general(m8, counts, (((1,), (0,)), ((), ())),
                                precision=lax.Precision.HIGHEST,
                                preferred_element_type=jnp.float32)
        ids8 = lax.broadcasted_iota(jnp.int32, (E, 1), 0)
        cmin = jnp.min(counts, axis=0, keepdims=True)
        emin = jnp.min(jnp.where(counts == cmin, ids8, E), axis=0,
                       keepdims=True)
        zslot = emin * CAP + (CAP - 1)
        pad = lax.broadcasted_iota(jnp.int32, (16, 1), 0)
        padv = jnp.where(pad == 0, zslot, 0)
        meta_ref[...] = jnp.concatenate(
            [start.astype(jnp.int32), counts.astype(jnp.int32), padv], axis=0)
        fp = jnp.sum((counts / N) * (psum[...] / N), axis=(0, 1), keepdims=True)
        aux_ref[...] = AUX_COEF * E * fp + Z_COEF * (zsum[...] / N)


def _router(x_flat, gate_w):
    return pl.pallas_call(
        _router_body,
        grid=(NB,),
        in_specs=[
            pl.BlockSpec((TB, D), lambda b: (b, 0)),
            pl.BlockSpec((E, D), lambda b: (0, 0)),
        ],
        out_specs=[
            pl.BlockSpec((None, 1, TB), lambda b: (b, 0, 0)),
            pl.BlockSpec((None, 1, TB), lambda b: (b, 0, 0)),
            pl.BlockSpec((None, 1, TB), lambda b: (b, 0, 0)),
            pl.BlockSpec((32, 1), lambda b: (0, 0)),
            pl.BlockSpec((1, 1), lambda b: (0, 0)),
        ],
        out_shape=[
            jax.ShapeDtypeStruct((NB, 1, TB), jnp.int32),
            jax.ShapeDtypeStruct((NB, 1, TB), jnp.int32),
            jax.ShapeDtypeStruct((NB, 1, TB), jnp.float32),
            jax.ShapeDtypeStruct((32, 1), jnp.int32),
            jax.ShapeDtypeStruct((1, 1), jnp.float32),
        ],
        scratch_shapes=[
            pltpu.VMEM((E, 1), jnp.float32),
            pltpu.VMEM((E, 1), jnp.float32),
            pltpu.VMEM((1, 1), jnp.float32),
        ],
    )(x_flat, gate_w)


def _sc_wid():
    return lax.axis_index("s") * 2 + lax.axis_index("c")


def _scatter_body(eidx_hbm, rank_hbm, p_hbm, meta_hbm,
                  sidx_hbm, q_hbm, g_hbm,
                  ev, rv, pv, metav, spv, tokv, gv, sem):
    wid = _sc_wid()
    base = wid * TW
    d0 = pltpu.async_copy(eidx_hbm.at[pl.ds(base, TW)], ev, sem)
    d1 = pltpu.async_copy(rank_hbm.at[pl.ds(base, TW)], rv, sem)
    d2 = pltpu.async_copy(p_hbm.at[pl.ds(base, TW)], pv, sem)
    d3 = pltpu.async_copy(meta_hbm, metav, sem)
    d0.wait(); d1.wait(); d2.wait(); d3.wait()

    iota = lax.iota(jnp.int32, 16)
    z16 = jnp.full((16,), 16, jnp.int32)
    zs16 = plsc.load_gather(metav, [z16])
    for i in range(TW // 16):
        sl = pl.ds(i * 16, 16)
        e16 = ev[sl]
        r16 = rv[sl]
        st16 = plsc.load_gather(metav, [e16])
        sp16 = st16 + r16
        c16 = e16 * CAP + r16
        gv[sl] = jnp.where(r16 < CAP, c16, zs16)
        spv[i // 8, pl.ds((i % 8) * 16, 16)] = sp16
        tokv[sl] = base + i * 16 + iota

    w0 = pltpu.async_copy(gv, g_hbm.at[pl.ds(base, TW)], sem)
    ws = []
    for j in range(TW // 128):
        ws.append(pltpu.async_copy(tokv.at[pl.ds(j * 128, 128)],
                                   sidx_hbm.at[spv.at[j]], sem))
        ws.append(pltpu.async_copy(pv.at[pl.ds(j * 128, 128)],
                                   q_hbm.at[spv.at[j]], sem))
    w0.wait()
    for w in ws:
        w.wait()


def _sc_scatter(eidx, rank, p, meta):
    f = pl.kernel(
        _scatter_body,
        out_type=[
            jax.ShapeDtypeStruct((N,), jnp.int32),
            jax.ShapeDtypeStruct((N,), jnp.float32),
            jax.ShapeDtypeStruct((N,), jnp.int32),
        ],
        mesh=plsc.VectorSubcoreMesh(core_axis_name="c", subcore_axis_name="s", num_cores=2, num_subcores=16),
        compiler_params=pltpu.CompilerParams(needs_layout_passes=False),
        scratch_types=[
            pltpu.VMEM((TW,), jnp.int32),
            pltpu.VMEM((TW,), jnp.int32),
            pltpu.VMEM((TW,), jnp.float32),
            pltpu.VMEM((32,), jnp.int32),
            pltpu.VMEM((TW // 128, 128), jnp.int32),
            pltpu.VMEM((TW,), jnp.int32),
            pltpu.VMEM((TW,), jnp.int32),
            pltpu.SemaphoreType.DMA,
        ],
    )
    return f(eidx, rank, p, meta)


def _dispatch_body(x_hbm, sidx_hbm, q_hbm, meta_hbm,
                   xd_hbm, scale_hbm,
                   metav, jv, tv, qv, scv, rb0, rb1, sem, gs0, gs1):
    wid = _sc_wid()
    slot_base = wid * TW
    e = slot_base // CAP
    rbase = slot_base - e * CAP
    pltpu.sync_copy(meta_hbm, metav)

    iota = lax.iota(jnp.int32, 16)
    e16 = jnp.zeros((16,), jnp.int32) + e
    st16 = plsc.load_gather(metav, [e16])
    cn16 = plsc.load_gather(metav, [e16 + 8])
    for i in range(TW // 16):
        sl = pl.ds(i * 16, 16)
        r16 = rbase + i * 16 + iota
        jm = st16 + jnp.minimum(r16, jnp.maximum(cn16 - 1, 0))
        jv[sl] = jnp.minimum(jm, N - 1)

    for j in range(TW // 128):
        sl = pl.ds(j * 128, 128)
        pltpu.sync_copy(sidx_hbm.at[jv.at[sl]], tv.at[sl])
    for j in range(TW // 128):
        sl = pl.ds(j * 128, 128)
        pltpu.sync_copy(q_hbm.at[tv.at[sl]], qv.at[sl])

    for i in range(TW // 16):
        sl = pl.ds(i * 16, 16)
        r16 = rbase + i * 16 + iota
        scv[sl] = jnp.where(r16 < cn16, qv[sl], 0.0)
    wsc = pltpu.async_copy(scv, scale_hbm.at[pl.ds(slot_base, TW)], sem)

    bufs = (rb0, rb1)
    sems = (gs0, gs1)
    descs = [None, None]
    descs[0] = pltpu.async_copy(x_hbm.at[tv.at[pl.ds(0, RCH)]], bufs[0], gs0)
    for ch in range(NCH):
        nxt = ch + 1
        if nxt < NCH:
            descs[nxt % 2] = pltpu.async_copy(
                x_hbm.at[tv.at[pl.ds(nxt * RCH, RCH)]], bufs[nxt % 2],
                sems[nxt % 2])
        descs[ch % 2].wait()
        pltpu.sync_copy(bufs[ch % 2],
                        xd_hbm.at[pl.ds(slot_base + ch * RCH, RCH)])
    wsc.wait()


def _sc_dispatch(x_bf, sidx, q, meta):
    f = pl.kernel(
        _dispatch_body,
        out_type=[
            jax.ShapeDtypeStruct((N, D), jnp.float32),
            jax.ShapeDtypeStruct((N,), jnp.float32),
        ],
        mesh=plsc.VectorSubcoreMesh(core_axis_name="c", subcore_axis_name="s", num_cores=2, num_subcores=16),
        compiler_params=pltpu.CompilerParams(needs_layout_passes=False),
        scratch_types=[
            pltpu.VMEM((32,), jnp.int32),
            pltpu.VMEM((TW,), jnp.int32),
            pltpu.VMEM((TW,), jnp.int32),
            pltpu.VMEM((TW,), jnp.float32),
            pltpu.VMEM((TW,), jnp.float32),
            pltpu.VMEM((RCH, D), jnp.float32),
            pltpu.VMEM((RCH, D), jnp.float32),
            pltpu.SemaphoreType.DMA,
            pltpu.SemaphoreType.DMA,
            pltpu.SemaphoreType.DMA,
        ],
    )
    return f(x_bf, sidx, q, meta)


def _gelu(v):
    return 0.5 * v * (1.0 + lax.erf(v * (1.0 / math.sqrt(2.0))))


def _ffn_body(xd_ref, w1_ref, b1_ref, w2_ref, b2_ref, sc_ref, out_ref,
              xbf, acc):
    h = pl.program_id(1)

    @pl.when(h == 0)
    def _():
        xbf[...] = xd_ref[...].astype(jnp.bfloat16)
        acc[...] = jnp.broadcast_to(b2_ref[0], (CAP, D))

    hm = lax.dot_general(xbf[...], w1_ref[...], (((1,), (1,)), ((), ())),
                         preferred_element_type=jnp.float32)
    hm = hm + b1_ref[0]
    hg = _gelu(hm).astype(jnp.bfloat16)
    acc[...] += lax.dot_general(hg, w2_ref[...], (((1,), (1,)), ((), ())),
                                preferred_element_type=jnp.float32)

    @pl.when(h == HK - 1)
    def _():
        out_ref[...] = acc[...] * sc_ref[...]


def _ffn(xd, w1, b1, w2, b2, scale):
    return pl.pallas_call(
        _ffn_body,
        grid=(E, HK),
        in_specs=[
            pl.BlockSpec((None, CAP, D), lambda e, h: (e, 0, 0)),
            pl.BlockSpec((None, HB, D), lambda e, h: (e, h, 0)),
            pl.BlockSpec((None, 1, HB), lambda e, h: (e, 0, h)),
            pl.BlockSpec((None, D, HB), lambda e, h: (e, 0, h)),
            pl.BlockSpec((None, 1, D), lambda e, h: (e, 0, 0)),
            pl.BlockSpec((None, CAP, 1), lambda e, h: (e, 0, 0)),
        ],
        out_specs=pl.BlockSpec((None, CAP, D), lambda e, h: (e, 0, 0)),
        out_shape=jax.ShapeDtypeStruct((E, CAP, D), jnp.float32),
        compiler_params=pltpu.CompilerParams(vmem_limit_bytes=117440512),
        scratch_shapes=[
            pltpu.VMEM((CAP, D), jnp.bfloat16),
            pltpu.VMEM((CAP, D), jnp.float32),
        ],
    )(xd, w1, b1, w2, b2, scale)


RC5 = 32
NC5 = TW // RC5


def _combine_body(od_hbm, g_hbm, out_hbm, gv, rb0, rb1, gs0, gs1):
    wid = _sc_wid()
    base = wid * TW
    pltpu.sync_copy(g_hbm.at[pl.ds(base, TW)], gv)
    bufs = (rb0, rb1)
    sems = (gs0, gs1)
    descs = [None, None]
    descs[0] = pltpu.async_copy(od_hbm.at[gv.at[pl.ds(0, RC5)]], bufs[0], gs0)
    for ch in range(NC5):
        nxt = ch + 1
        if nxt < NC5:
            descs[nxt % 2] = pltpu.async_copy(
                od_hbm.at[gv.at[pl.ds(nxt * RC5, RC5)]], bufs[nxt % 2],
                sems[nxt % 2])
        descs[ch % 2].wait()
        pltpu.sync_copy(bufs[ch % 2], out_hbm.at[pl.ds(base + ch * RC5, RC5)])


def _sc_combine(od, g):
    f = pl.kernel(
        _combine_body,
        out_type=jax.ShapeDtypeStruct((N, D), jnp.float32),
        mesh=plsc.VectorSubcoreMesh(core_axis_name="c", subcore_axis_name="s", num_cores=2, num_subcores=16),
        compiler_params=pltpu.CompilerParams(needs_layout_passes=False),
        scratch_types=[
            pltpu.VMEM((TW,), jnp.int32),
            pltpu.VMEM((RC5, D), jnp.float32),
            pltpu.VMEM((RC5, D), jnp.float32),
            pltpu.SemaphoreType.DMA,
            pltpu.SemaphoreType.DMA,
        ],
    )
    return f(od, g)


@jax.jit
def kernel(x, gate_w, W1, b1, W2, b2):
    x_flat = x.reshape(N, D)
    eidx, rank, p, meta, aux = _router(x_flat, gate_w)
    sidx, q, g = _sc_scatter(eidx.reshape(N), rank.reshape(N),
                             p.reshape(N), meta.reshape(32))
    x_disp, scale = _sc_dispatch(x_flat, sidx, q, meta.reshape(32))
    od = _ffn(x_disp.reshape(E, CAP, D),
              W1.astype(jnp.bfloat16), b1.reshape(E, 1, H),
              W2.astype(jnp.bfloat16), b2.reshape(E, 1, D),
              scale.reshape(E, CAP, 1))
    out = _sc_combine(od.reshape(N, D), g)
    return out.reshape(B, T, D), aux.reshape(())[()]
